# speedup vs baseline: 5.8832x; 5.8832x over previous
import functools

import jax
import jax.numpy as jnp
from jax import lax
from jax.experimental import pallas as pl
from jax.experimental.pallas import tpu as pltpu
from jax.experimental.pallas import tpu_sc as plsc

EMB = 128
INTER = 256


def _sc_gather(qids, eids, qtab, etab):
    info = plsc.get_sparse_core_info()
    nw = info.num_cores * info.num_subcores
    bq = qids.shape[0]
    be = eids.shape[0]
    qw = bq // nw
    ew = be // nw
    c = 128
    mesh = plsc.VectorSubcoreMesh(core_axis_name="c", subcore_axis_name="s")

    @functools.partial(
        pl.kernel,
        mesh=mesh,
        out_type=(
            jax.ShapeDtypeStruct((bq, EMB), jnp.float32),
            jax.ShapeDtypeStruct((be, EMB), jnp.float32),
        ),
        scratch_types=[
            pltpu.VMEM((c,), jnp.int32),
            pltpu.VMEM((c, EMB), jnp.float32),
            pltpu.SemaphoreType.DMA,
        ],
    )
    def k(qids_h, eids_h, qtab_h, etab_h, qout_h, eout_h, idx_v, rows_v, sem):
        wid = lax.axis_index("s") * info.num_cores + lax.axis_index("c")
        qbase = wid * qw
        for j in range(qw // c):
            b = qbase + j * c
            pltpu.sync_copy(qids_h.at[pl.ds(b, c)], idx_v)
            pltpu.async_copy(qtab_h.at[idx_v], rows_v, sem).wait()
            pltpu.sync_copy(rows_v, qout_h.at[pl.ds(b, c)])
        ebase = wid * ew
        for j in range(ew // c):
            b = ebase + j * c
            pltpu.sync_copy(eids_h.at[pl.ds(b, c)], idx_v)
            pltpu.async_copy(etab_h.at[idx_v], rows_v, sem).wait()
            pltpu.sync_copy(rows_v, eout_h.at[pl.ds(b, c)])

    return k(qids, eids, qtab, etab)


def _mlp_body(x_ref, w1_ref, b1_ref, w2_ref, b2_ref, o_ref):
    h = jnp.dot(x_ref[...], w1_ref[...], preferred_element_type=jnp.float32)
    h = jax.nn.gelu(h + b1_ref[...])
    o_ref[...] = jnp.dot(h, w2_ref[...], preferred_element_type=jnp.float32) + b2_ref[...]


def _tc_mlp(x, w1, b1, w2, b2, blk):
    n = x.shape[0]
    return pl.pallas_call(
        _mlp_body,
        grid=(n // blk,),
        in_specs=[
            pl.BlockSpec((blk, EMB), lambda i: (i, 0)),
            pl.BlockSpec((EMB, INTER), lambda i: (0, 0)),
            pl.BlockSpec((1, INTER), lambda i: (0, 0)),
            pl.BlockSpec((INTER, EMB), lambda i: (0, 0)),
            pl.BlockSpec((1, EMB), lambda i: (0, 0)),
        ],
        out_specs=pl.BlockSpec((blk, EMB), lambda i: (i, 0)),
        out_shape=jax.ShapeDtypeStruct((n, EMB), jnp.float32),
    )(x, w1, b1.reshape(1, -1), w2, b2.reshape(1, -1))


def kernel(query_ids, entity_ids, ent_table, query_table, W1, b1, W2, b2):
    q_rows, e_rows = _sc_gather(query_ids, entity_ids, query_table, ent_table)
    q_out = _tc_mlp(q_rows, W1, b1, W2, b2, blk=512)
    e_out = _tc_mlp(e_rows, W1, b1, W2, b2, blk=512)
    return (q_out, e_out)


# speedup vs baseline: 6.3738x; 1.0834x over previous
import functools

import jax
import jax.numpy as jnp
from jax import lax
from jax.experimental import pallas as pl
from jax.experimental.pallas import tpu as pltpu
from jax.experimental.pallas import tpu_sc as plsc

EMB = 128
INTER = 256


def _sc_gather(qids, eids, qtab, etab):
    info = plsc.get_sparse_core_info()
    nw = info.num_cores * info.num_subcores
    bq = qids.shape[0]
    be = eids.shape[0]
    qw = bq // nw
    ew = be // nw
    c = 128
    mesh = plsc.VectorSubcoreMesh(core_axis_name="c", subcore_axis_name="s")

    @functools.partial(
        pl.kernel,
        mesh=mesh,
        out_type=(
            jax.ShapeDtypeStruct((bq, EMB), jnp.float32),
            jax.ShapeDtypeStruct((be, EMB), jnp.float32),
        ),
        scratch_types=[
            pltpu.VMEM((c,), jnp.int32),
            pltpu.VMEM((c,), jnp.int32),
            pltpu.VMEM((c, EMB), jnp.float32),
            pltpu.VMEM((c, EMB), jnp.float32),
            pltpu.SemaphoreType.DMA,
            pltpu.SemaphoreType.DMA,
            pltpu.SemaphoreType.DMA,
            pltpu.SemaphoreType.DMA,
        ],
    )
    def k(qids_h, eids_h, qtab_h, etab_h, qout_h, eout_h,
          idx0, idx1, rows0, rows1, g0, g1, w0, w1):
        wid = lax.axis_index("s") * info.num_cores + lax.axis_index("c")
        idx = (idx0, idx1)
        rows = (rows0, rows1)
        gsem = (g0, g1)
        wsem = (w0, w1)
        qbase = wid * qw
        ebase = wid * ew
        tasks = [(qids_h, qtab_h, qout_h, qbase + j * c) for j in range(qw // c)]
        tasks += [(eids_h, etab_h, eout_h, ebase + j * c) for j in range(ew // c)]
        n = len(tasks)

        def start_gather(t):
            buf = t % 2
            ids_h, tab_h, _, b = tasks[t]
            pltpu.sync_copy(ids_h.at[pl.ds(b, c)], idx[buf])
            return pltpu.async_copy(tab_h.at[idx[buf]], rows[buf], gsem[buf])

        gathers = {0: start_gather(0)}
        writebacks = {}
        for t in range(n):
            buf = t % 2
            if t + 1 < n:
                if t - 1 >= 0:
                    writebacks[t - 1].wait()
                gathers[t + 1] = start_gather(t + 1)
            gathers[t].wait()
            _, _, out_h, b = tasks[t]
            writebacks[t] = pltpu.async_copy(rows[buf], out_h.at[pl.ds(b, c)],
                                             wsem[buf])
        writebacks[n - 2].wait()
        writebacks[n - 1].wait()

    return k(qids, eids, qtab, etab)


def _mlp_body(x_ref, w1_ref, b1_ref, w2_ref, b2_ref, o_ref):
    h = jnp.dot(x_ref[...], w1_ref[...], preferred_element_type=jnp.float32)
    h = jax.nn.gelu(h + b1_ref[...])
    o_ref[...] = jnp.dot(h, w2_ref[...], preferred_element_type=jnp.float32) + b2_ref[...]


def _tc_mlp(x, w1, b1, w2, b2, blk):
    n = x.shape[0]
    return pl.pallas_call(
        _mlp_body,
        grid=(n // blk,),
        in_specs=[
            pl.BlockSpec((blk, EMB), lambda i: (i, 0)),
            pl.BlockSpec((EMB, INTER), lambda i: (0, 0)),
            pl.BlockSpec((1, INTER), lambda i: (0, 0)),
            pl.BlockSpec((INTER, EMB), lambda i: (0, 0)),
            pl.BlockSpec((1, EMB), lambda i: (0, 0)),
        ],
        out_specs=pl.BlockSpec((blk, EMB), lambda i: (i, 0)),
        out_shape=jax.ShapeDtypeStruct((n, EMB), jnp.float32),
    )(x, w1, b1.reshape(1, -1), w2, b2.reshape(1, -1))


def kernel(query_ids, entity_ids, ent_table, query_table, W1, b1, W2, b2):
    q_rows, e_rows = _sc_gather(query_ids, entity_ids, query_table, ent_table)
    q_out = _tc_mlp(q_rows, W1, b1, W2, b2, blk=512)
    e_out = _tc_mlp(e_rows, W1, b1, W2, b2, blk=512)
    return (q_out, e_out)
